# Initial kernel scaffold; baseline (speedup 1.0000x reference)
import jax
import jax.numpy as jnp
from jax.experimental import pallas as pl


def kernel(data, table):
    raise NotImplementedError("write your pallas kernel here")


# speedup vs baseline: 1.0422x; 1.0422x over previous
import functools

import jax
import jax.numpy as jnp
from jax import lax
from jax.experimental import pallas as pl
from jax.experimental.pallas import tpu as pltpu
from jax.experimental.pallas import tpu_sc as plsc

NUM_EMBEDDINGS = 1_000_000
EMBEDDING_DIM = 32
TOTAL_TOKENS = 819_200

NC = 2
NS = 16
NW = NC * NS
BPW = TOTAL_TOKENS // NW
GRP = 128
CH_G = 8
CH = CH_G * GRP
NCH = BPW // CH


def _sc_gather(data2d, table):
    mesh = plsc.VectorSubcoreMesh(core_axis_name="c", subcore_axis_name="s")

    @functools.partial(
        pl.kernel,
        mesh=mesh,
        out_type=jax.ShapeDtypeStruct((TOTAL_TOKENS, EMBEDDING_DIM), jnp.float32),
        scratch_types=[
            pltpu.VMEM((CH_G, GRP), jnp.int32),
            pltpu.VMEM((CH, EMBEDDING_DIM), jnp.float32),
            pltpu.SemaphoreType.DMA,
        ],
        compiler_params=pltpu.CompilerParams(use_tc_tiling_on_sc=False),
    )
    def k(idx_hbm, table_hbm, out_hbm, idx_v, rows_v, sem):
        wid = lax.axis_index("s") * NC + lax.axis_index("c")
        gbase = wid * (BPW // GRP)

        @pl.loop(0, NCH)
        def body(g):
            goff = gbase + g * CH_G
            pltpu.sync_copy(idx_hbm.at[pl.ds(goff, CH_G)], idx_v)
            copies = [
                pltpu.async_copy(
                    table_hbm.at[idx_v.at[j]],
                    rows_v.at[pl.ds(j * GRP, GRP)],
                    sem,
                )
                for j in range(CH_G)
            ]
            for c in copies:
                c.wait()
            pltpu.sync_copy(rows_v, out_hbm.at[pl.ds(goff * GRP, CH)])

    return k(data2d, table)


@jax.jit
def kernel(data, table):
    data2d = data.reshape(TOTAL_TOKENS // GRP, GRP)
    return _sc_gather(data2d, table)
